# baseline (speedup 1.0000x reference)
import jax
import jax.numpy as jnp
from jax.experimental import pallas as pl
from jax.experimental.pallas import tpu as pltpu

_BM = 2048
_BK = 200


def _matmul_body(xt_ref, w_ref, o_ref):
    x = xt_ref[...].astype(jnp.bfloat16)
    w = w_ref[...].astype(jnp.bfloat16)
    acc = jax.lax.dot_general(
        x, w, (((0,), (0,)), ((), ())),
        preferred_element_type=jnp.float32)
    j = pl.program_id(1)

    @pl.when(j == 0)
    def _init():
        o_ref[...] = acc

    @pl.when(j != 0)
    def _accum():
        o_ref[...] += acc


def kernel(inputs, kernel):
    m, k = inputs.shape
    _, n = kernel.shape
    bm = min(_BM, m)
    bk = _BK if k % _BK == 0 else k
    xt = inputs.T
    return pl.pallas_call(
        _matmul_body,
        grid=(m // bm, k // bk),
        in_specs=[
            pl.BlockSpec((bk, bm), lambda i, j: (j, i)),
            pl.BlockSpec((bk, n), lambda i, j: (j, 0)),
        ],
        out_specs=pl.BlockSpec((bm, n), lambda i, j: (i, 0)),
        out_shape=jax.ShapeDtypeStruct((m, n), jnp.float32),
        compiler_params=pltpu.CompilerParams(
            dimension_semantics=("parallel", "arbitrary"),
        ),
    )(xt, kernel)


# speedup vs baseline: 1.8285x; 1.8285x over previous
import jax
import jax.numpy as jnp
from jax.experimental import pallas as pl
from jax.experimental.pallas import tpu as pltpu

_BM = 2048


def _matmul_body(x1_ref, x2_ref, w_ref, o_ref):
    w = w_ref[...].astype(jnp.bfloat16)
    half = o_ref.shape[0] // 2
    x1 = x1_ref[...].astype(jnp.bfloat16)
    o_ref[:half, :] = jax.lax.dot_general(
        x1, w, (((0,), (0,)), ((), ())),
        preferred_element_type=jnp.float32)
    x2 = x2_ref[...].astype(jnp.bfloat16)
    o_ref[half:, :] = jax.lax.dot_general(
        x2, w, (((0,), (0,)), ((), ())),
        preferred_element_type=jnp.float32)


def kernel(inputs, kernel):
    m, k = inputs.shape
    _, n = kernel.shape
    bm = min(_BM, m)
    xt = inputs.T
    return pl.pallas_call(
        _matmul_body,
        grid=(m // bm,),
        in_specs=[
            pl.BlockSpec((k, bm // 2), lambda i: (0, 2 * i)),
            pl.BlockSpec((k, bm // 2), lambda i: (0, 2 * i + 1)),
            pl.BlockSpec((k, n), lambda i: (0, 0)),
        ],
        out_specs=pl.BlockSpec((bm, n), lambda i: (i, 0)),
        out_shape=jax.ShapeDtypeStruct((m, n), jnp.float32),
        compiler_params=pltpu.CompilerParams(
            dimension_semantics=("parallel",),
        ),
    )(xt, xt, kernel)
